# baseline (speedup 1.0000x reference)
import jax
import jax.numpy as jnp
from jax.experimental import pallas as pl
from jax.experimental.pallas import tpu as pltpu

_B = 32
_C = 192
_H = 56
_W = 56
_N = _C * _H * _W
_K = int(round(_N * 0.1))
_BOOST_STRENGTH = 1.0
_S = 2
_NCHUNK = 12
_CP = _C // 2 // _NCHUNK


def _key_to_f32(m):
    return jax.lax.bitcast_convert_type(
        m ^ ((m >> 31) & jnp.int32(0x7FFFFFFF)), jnp.float32)


def _body(x_ref, bf_ref, out_ref, pk_ref):
    bf = bf_ref[...]
    kf = jnp.float32(_K)
    nf = jnp.float32(_N)
    for s in range(_S):
        b = x_ref[s] * bf
        pk_ref[s] = jnp.concatenate([b[:_C // 2], b[_C // 2:]], axis=2)

    def count_ge(s, fmid):
        parts = []
        for g in range(_NCHUNK):
            blk = pk_ref[s, g * _CP:(g + 1) * _CP]
            m = jnp.where(blk >= fmid, jnp.float32(1.0), jnp.float32(0.0))
            parts.append(jnp.sum(m, axis=(0, 1)))
        while len(parts) > 1:
            nxt = [a + b for a, b in zip(parts[0::2], parts[1::2])]
            if len(parts) % 2:
                nxt.append(parts[-1])
            parts = nxt
        return jnp.sum(parts[0])

    def cond(carry):
        open_any = None
        for s in range(_S):
            lo, hi = carry[4 * s], carry[4 * s + 1]
            o = lo < hi - jnp.int32(1)
            open_any = o if open_any is None else (open_any | o)
        return open_any

    def step(carry):
        it = carry[-1]
        nxt = []
        for s in range(_S):
            lo, hi, clo, chi = carry[4 * s:4 * s + 4]
            active = lo < hi - jnp.int32(1)
            bis = (lo & hi) + ((lo ^ hi) >> 1)
            frac = (clo - kf) / jnp.maximum(clo - chi, jnp.float32(1.0))
            midf = jnp.float32(lo) + (jnp.float32(hi) - jnp.float32(lo)) * frac
            midf = jnp.clip(midf, jnp.float32(lo) + 1.0, jnp.float32(hi) - 1.0)
            interp = jnp.clip(midf.astype(jnp.int32), lo + jnp.int32(1),
                              hi - jnp.int32(1))
            mid = jnp.where(it % 2 == 0, interp, bis)
            cnt = count_ge(s, _key_to_f32(mid))
            ok = cnt >= kf
            done = cnt == kf
            nlo = jnp.where(ok, mid, lo)
            nclo = jnp.where(ok, cnt, clo)
            nhi = jnp.where(done, mid + jnp.int32(1), jnp.where(ok, hi, mid))
            nchi = jnp.where(ok, chi, cnt)
            nxt.append(jnp.where(active, nlo, lo))
            nxt.append(jnp.where(active, nhi, hi))
            nxt.append(jnp.where(active, nclo, clo))
            nxt.append(jnp.where(active, nchi, chi))
        nxt.append(it + jnp.int32(1))
        return tuple(nxt)

    lo_inf = jnp.int32(-2139095041)
    p0 = jnp.int32(0x3F8CCCCD)
    init = []
    for s in range(_S):
        bmax = jnp.max(pk_ref[s])
        imax = jax.lax.bitcast_convert_type(bmax, jnp.int32)
        hi0 = (imax ^ ((imax >> 31) & jnp.int32(0x7FFFFFFF))) + jnp.int32(1)
        c0 = count_ge(s, jnp.float32(1.1))
        ok0 = c0 >= kf
        in_rng = p0 < hi0
        lo1 = jnp.where(ok0 & in_rng, p0, lo_inf)
        clo1 = jnp.where(ok0 & in_rng, c0, nf)
        hi1 = jnp.where((~ok0) & in_rng, p0, hi0)
        chi1 = jnp.where((~ok0) & in_rng, c0, jnp.float32(0.0))
        done0 = (c0 == kf) & in_rng
        hi1 = jnp.where(done0, p0 + jnp.int32(1), hi1)
        lo1 = jnp.where(done0, p0, lo1)
        init += [lo1, hi1, clo1, chi1]
    init.append(jnp.int32(0))
    res = jax.lax.while_loop(cond, step, tuple(init))
    for s in range(_S):
        ft = _key_to_f32(res[4 * s])
        xs = x_ref[s]
        out_ref[s] = jnp.where(xs * bf >= ft, xs, jnp.float32(0.0))


def kernel(x, dutyCycle):
    target_density = jnp.float32(float(_K) / float(_N))
    bf = jnp.exp((target_density - dutyCycle.reshape(_C)) * jnp.float32(_BOOST_STRENGTH))
    bf_full = jnp.broadcast_to(bf[:, None, None], (_C, _H, _W))
    return pl.pallas_call(
        _body,
        grid=(_B // _S,),
        in_specs=[
            pl.BlockSpec((_S, _C, _H, _W), lambda b: (b, 0, 0, 0)),
            pl.BlockSpec((_C, _H, _W), lambda b: (0, 0, 0)),
        ],
        out_specs=pl.BlockSpec((_S, _C, _H, _W), lambda b: (b, 0, 0, 0)),
        out_shape=jax.ShapeDtypeStruct((_B, _C, _H, _W), jnp.float32),
        scratch_shapes=[pltpu.VMEM((_S, _C // 2, _H, 2 * _W), jnp.float32)],
    )(x, bf_full)
